# baseline (speedup 1.0000x reference)
import functools

import jax
import jax.numpy as jnp
from jax import lax
from jax.experimental import pallas as pl
from jax.experimental.pallas import tpu as pltpu
from jax.experimental.pallas import tpu_sc as plsc

N = 1024
C = 256
PH = PW = 7
SPB = 64
GPB = PH
SPG = 8
NS = GPB * SPG
ROWS = N * NS
NW = 32
RPW = ROWS // NW
GPW = N * GPB // NW
G = 32
NCH = RPW // G
LEVEL_ROWS = (65536, 16384, 4096, 1024)
TABLE_ROWS = sum(LEVEL_ROWS)


def _index_kernel(meta_ref, boxes_ref, i00, i01, i10, i11, w00, w01, w10, w11):
    rows = meta_ref[0, 0]
    cols = meta_ref[0, 1]
    b = boxes_ref[...]
    x1 = b[:, 0:1]; y1 = b[:, 1:2]; x2 = b[:, 2:3]; y2 = b[:, 3:4]
    h = y2 - y1
    w = x2 - x1
    image_area = rows * cols
    roi_level = jnp.log(jnp.sqrt(h * w) / jnp.sqrt(image_area)) / jnp.log(2.0)
    roi_level = jnp.minimum(5.0, jnp.maximum(2.0, 4.0 + jnp.round(roi_level)))
    lvl = roi_level.astype(jnp.int32) - 2
    side = lax.shift_right_logical(jnp.full_like(lvl, 256), lvl)
    base = jnp.where(
        lvl == 0, 0,
        jnp.where(lvl == 1, LEVEL_ROWS[0],
                  jnp.where(lvl == 2, LEVEL_ROWS[0] + LEVEL_ROWS[1],
                            LEVEL_ROWS[0] + LEVEL_ROWS[1] + LEVEL_ROWS[2])))
    sm1 = (side - 1).astype(jnp.float32)
    x1n = x1 / (cols - 1.0); x2n = x2 / (cols - 1.0)
    y1n = y1 / (rows - 1.0); y2n = y2 / (rows - 1.0)

    s = lax.broadcasted_iota(jnp.int32, (N, SPB), 1)
    iy = jnp.minimum(s // SPG, PH - 1).astype(jnp.float32) / float(PH - 1)
    ix = jnp.minimum(s % SPG, PW - 1).astype(jnp.float32) / float(PW - 1)
    ys = (y1n + iy * (y2n - y1n)) * sm1
    xs = (x1n + ix * (x2n - x1n)) * sm1
    y0f = jnp.floor(ys); x0f = jnp.floor(xs)
    y0 = jnp.clip(y0f, 0, sm1).astype(jnp.int32)
    y1c = jnp.clip(y0f + 1.0, 0, sm1).astype(jnp.int32)
    x0 = jnp.clip(x0f, 0, sm1).astype(jnp.int32)
    x1c = jnp.clip(x0f + 1.0, 0, sm1).astype(jnp.int32)
    wy = jnp.clip(ys - y0f, 0.0, 1.0)
    wx = jnp.clip(xs - x0f, 0.0, 1.0)

    rbase = base + y0 * side
    rbase1 = base + y1c * side
    i00[...] = rbase + x0
    i01[...] = rbase + x1c
    i10[...] = rbase1 + x0
    i11[...] = rbase1 + x1c
    w00[...] = (1.0 - wy) * (1.0 - wx)
    w01[...] = (1.0 - wy) * wx
    w10[...] = wy * (1.0 - wx)
    w11[...] = wy * wx


def _compute_indices(metadata, boxes2d, interpret=False):
    shp_i = jax.ShapeDtypeStruct((N, SPB), jnp.int32)
    shp_f = jax.ShapeDtypeStruct((N, SPB), jnp.float32)
    return pl.pallas_call(
        _index_kernel,
        out_shape=(shp_i, shp_i, shp_i, shp_i, shp_f, shp_f, shp_f, shp_f),
        interpret=interpret,
    )(metadata, boxes2d)


def _sc_gather_body(table, i00, i01, i10, i11, w00, w01, w10, w11, out,
                    i00v, i01v, i10v, i11v, w00v, w01v, w10v, w11v,
                    r00a, r01a, r10a, r11a, obufa,
                    r00b, r01b, r10b, r11b, obufb,
                    gsem0, gsem1, wsem0, wsem1):
    wid = lax.axis_index("s") * 2 + lax.axis_index("c")
    wbase = wid * RPW
    pltpu.sync_copy(i00.at[pl.ds(wbase, RPW)], i00v)
    pltpu.sync_copy(i01.at[pl.ds(wbase, RPW)], i01v)
    pltpu.sync_copy(i10.at[pl.ds(wbase, RPW)], i10v)
    pltpu.sync_copy(i11.at[pl.ds(wbase, RPW)], i11v)
    pltpu.sync_copy(w00.at[pl.ds(wbase, RPW)], w00v)
    pltpu.sync_copy(w01.at[pl.ds(wbase, RPW)], w01v)
    pltpu.sync_copy(w10.at[pl.ds(wbase, RPW)], w10v)
    pltpu.sync_copy(w11.at[pl.ds(wbase, RPW)], w11v)

    bufs = ((r00a, r01a, r10a, r11a, obufa, gsem0, wsem0),
            (r00b, r01b, r10b, r11b, obufb, gsem1, wsem1))

    dnums = lax.GatherDimensionNumbers(
        offset_dims=(), collapsed_slice_dims=(0,), start_index_map=(0,))

    def bcast(v16, g):
        idx = jnp.full((16, 1), g, jnp.int32)
        return lax.gather(v16, idx, dnums, (1,),
                          mode=lax.GatherScatterMode.PROMISE_IN_BOUNDS)

    def issue(i, s):
        lo = i * G
        r0, r1, r2, r3, _, gs, _ = bufs[s]
        pltpu.async_copy(table.at[i00v.at[pl.ds(lo, G)]], r0, gs)
        pltpu.async_copy(table.at[i01v.at[pl.ds(lo, G)]], r1, gs)
        pltpu.async_copy(table.at[i10v.at[pl.ds(lo, G)]], r2, gs)
        pltpu.async_copy(table.at[i11v.at[pl.ds(lo, G)]], r3, gs)

    def gdrain(s):
        r0, r1, r2, r3, _, gs, _ = bufs[s]
        for r in (r0, r1, r2, r3):
            pltpu.make_async_copy(table.at[pl.ds(0, G)], r, gs).wait()

    def wissue(i, s):
        _, _, _, _, ob, _, ws = bufs[s]
        pltpu.async_copy(ob, out.at[pl.ds(wbase + i * G, G)], ws)

    def wdrain(s):
        _, _, _, _, ob, _, ws = bufs[s]
        pltpu.make_async_copy(ob, out.at[pl.ds(0, G)], ws).wait()

    def compute(i, s):
        lo = i * G
        r0, r1, r2, r3, ob, _, _ = bufs[s]

        def group(q, _):
            gb = q * 16
            wa16 = w00v[pl.ds(lo + gb, 16)]
            wb16 = w01v[pl.ds(lo + gb, 16)]
            wc16 = w10v[pl.ds(lo + gb, 16)]
            wd16 = w11v[pl.ds(lo + gb, 16)]

            @plsc.parallel_loop(0, 16, unroll=2)
            def row(g):
                rr = gb + g
                a = bcast(wa16, g)
                bq = bcast(wb16, g)
                cq = bcast(wc16, g)
                dq = bcast(wd16, g)
                for c in range(C // 16):
                    sl = pl.ds(c * 16, 16)
                    ob[rr, sl] = (a * r0[rr, sl] + bq * r1[rr, sl]
                                  + cq * r2[rr, sl] + dq * r3[rr, sl])
            return 0

        lax.fori_loop(0, G // 16, group, 0, unroll=False)

    issue(0, 0)

    @pl.loop(0, NCH, step=2)
    def pair(i):
        issue(i + 1, 1)
        gdrain(0)
        pl.when(i > 0)(lambda: wdrain(0))
        compute(i, 0)
        wissue(i, 0)
        pl.when(i + 2 < NCH)(lambda: issue(i + 2, 0))
        gdrain(1)
        pl.when(i > 0)(lambda: wdrain(1))
        compute(i + 1, 1)
        wissue(i + 1, 1)

    wdrain(0)
    wdrain(1)


@functools.lru_cache(maxsize=None)
def _get_sc_gather():
    return pl.kernel(
        _sc_gather_body,
        out_type=jax.ShapeDtypeStruct((ROWS, C), jnp.float32),
        mesh=plsc.VectorSubcoreMesh(core_axis_name="c", subcore_axis_name="s"),
        scratch_types=[
            pltpu.VMEM((RPW,), jnp.int32),
            pltpu.VMEM((RPW,), jnp.int32),
            pltpu.VMEM((RPW,), jnp.int32),
            pltpu.VMEM((RPW,), jnp.int32),
            pltpu.VMEM((RPW,), jnp.float32),
            pltpu.VMEM((RPW,), jnp.float32),
            pltpu.VMEM((RPW,), jnp.float32),
            pltpu.VMEM((RPW,), jnp.float32),
            pltpu.VMEM((G, C), jnp.float32),
            pltpu.VMEM((G, C), jnp.float32),
            pltpu.VMEM((G, C), jnp.float32),
            pltpu.VMEM((G, C), jnp.float32),
            pltpu.VMEM((G, C), jnp.float32),
            pltpu.VMEM((G, C), jnp.float32),
            pltpu.VMEM((G, C), jnp.float32),
            pltpu.VMEM((G, C), jnp.float32),
            pltpu.VMEM((G, C), jnp.float32),
            pltpu.VMEM((G, C), jnp.float32),
            pltpu.SemaphoreType.DMA,
            pltpu.SemaphoreType.DMA,
            pltpu.SemaphoreType.DMA,
            pltpu.SemaphoreType.DMA,
        ],
    )


def kernel(metadata, boxes, feat_p2, feat_p3, feat_p4, feat_p5):
    boxes2d = boxes[0]
    idx_w = _compute_indices(metadata, boxes2d)
    flats = [f.reshape(-1, C) for f in (feat_p2, feat_p3, feat_p4, feat_p5)]
    table = jnp.concatenate(flats, axis=0)
    args = [a[:, :NS].reshape(ROWS) for a in idx_w]
    pooled = _get_sc_gather()(table, *args)
    return pooled.reshape(1, N, GPB, SPG, C)[:, :, :, :PW].reshape(1, N, PH, PW, C)


# speedup vs baseline: 1.0096x; 1.0096x over previous
import functools

import jax
import jax.numpy as jnp
from jax import lax
from jax.experimental import pallas as pl
from jax.experimental.pallas import tpu as pltpu
from jax.experimental.pallas import tpu_sc as plsc

N = 1024
C = 256
PH = PW = 7
SPB = 64
GPB = PH
SPG = 8
NS = GPB * SPG
ROWS = N * NS
NW = 32
RPW = ROWS // NW
GPW = N * GPB // NW
G = 32
NCH = RPW // G
LEVEL_ROWS = (65536, 16384, 4096, 1024)
TABLE_ROWS = sum(LEVEL_ROWS)


def _index_kernel(meta_ref, boxes_ref, i00, i01, i10, i11, w00, w01, w10, w11):
    rows = meta_ref[0, 0]
    cols = meta_ref[0, 1]
    b = boxes_ref[...]
    x1 = b[:, 0:1]; y1 = b[:, 1:2]; x2 = b[:, 2:3]; y2 = b[:, 3:4]
    h = y2 - y1
    w = x2 - x1
    image_area = rows * cols
    roi_level = jnp.log(jnp.sqrt(h * w) / jnp.sqrt(image_area)) / jnp.log(2.0)
    roi_level = jnp.minimum(5.0, jnp.maximum(2.0, 4.0 + jnp.round(roi_level)))
    lvl = roi_level.astype(jnp.int32) - 2
    side = lax.shift_right_logical(jnp.full_like(lvl, 256), lvl)
    base = jnp.where(
        lvl == 0, 0,
        jnp.where(lvl == 1, LEVEL_ROWS[0],
                  jnp.where(lvl == 2, LEVEL_ROWS[0] + LEVEL_ROWS[1],
                            LEVEL_ROWS[0] + LEVEL_ROWS[1] + LEVEL_ROWS[2])))
    sm1 = (side - 1).astype(jnp.float32)
    x1n = x1 / (cols - 1.0); x2n = x2 / (cols - 1.0)
    y1n = y1 / (rows - 1.0); y2n = y2 / (rows - 1.0)

    s = lax.broadcasted_iota(jnp.int32, (N, SPB), 1)
    iy = jnp.minimum(s // SPG, PH - 1).astype(jnp.float32) / float(PH - 1)
    ix = jnp.minimum(s % SPG, PW - 1).astype(jnp.float32) / float(PW - 1)
    ys = (y1n + iy * (y2n - y1n)) * sm1
    xs = (x1n + ix * (x2n - x1n)) * sm1
    y0f = jnp.floor(ys); x0f = jnp.floor(xs)
    y0 = jnp.clip(y0f, 0, sm1).astype(jnp.int32)
    y1c = jnp.clip(y0f + 1.0, 0, sm1).astype(jnp.int32)
    x0 = jnp.clip(x0f, 0, sm1).astype(jnp.int32)
    x1c = jnp.clip(x0f + 1.0, 0, sm1).astype(jnp.int32)
    wy = jnp.clip(ys - y0f, 0.0, 1.0)
    wx = jnp.clip(xs - x0f, 0.0, 1.0)

    rbase = base + y0 * side
    rbase1 = base + y1c * side
    i00[...] = rbase + x0
    i01[...] = rbase + x1c
    i10[...] = rbase1 + x0
    i11[...] = rbase1 + x1c
    w00[...] = (1.0 - wy) * (1.0 - wx)
    w01[...] = (1.0 - wy) * wx
    w10[...] = wy * (1.0 - wx)
    w11[...] = wy * wx


def _compute_indices(metadata, boxes2d, interpret=False):
    shp_i = jax.ShapeDtypeStruct((N, SPB), jnp.int32)
    shp_f = jax.ShapeDtypeStruct((N, SPB), jnp.float32)
    return pl.pallas_call(
        _index_kernel,
        out_shape=(shp_i, shp_i, shp_i, shp_i, shp_f, shp_f, shp_f, shp_f),
        interpret=interpret,
    )(metadata, boxes2d)


def _sc_gather_body(table, i00, i01, i10, i11, w00, w01, w10, w11, out,
                    i00v, i01v, i10v, i11v, w00v, w01v, w10v, w11v,
                    r00a, r01a, r10a, r11a, obufa,
                    r00b, r01b, r10b, r11b, obufb,
                    gsem0, gsem1, wsem0, wsem1):
    wid = lax.axis_index("s") * 2 + lax.axis_index("c")
    wbase = wid * RPW
    pltpu.sync_copy(i00.at[pl.ds(wbase, RPW)], i00v)
    pltpu.sync_copy(i01.at[pl.ds(wbase, RPW)], i01v)
    pltpu.sync_copy(i10.at[pl.ds(wbase, RPW)], i10v)
    pltpu.sync_copy(i11.at[pl.ds(wbase, RPW)], i11v)
    pltpu.sync_copy(w00.at[pl.ds(wbase, RPW)], w00v)
    pltpu.sync_copy(w01.at[pl.ds(wbase, RPW)], w01v)
    pltpu.sync_copy(w10.at[pl.ds(wbase, RPW)], w10v)
    pltpu.sync_copy(w11.at[pl.ds(wbase, RPW)], w11v)

    bufs = ((r00a, r01a, r10a, r11a, obufa, gsem0, wsem0),
            (r00b, r01b, r10b, r11b, obufb, gsem1, wsem1))

    dnums = lax.GatherDimensionNumbers(
        offset_dims=(), collapsed_slice_dims=(0,), start_index_map=(0,))

    def bcast(v16, g):
        idx = jnp.full((16, 1), g, jnp.int32)
        return lax.gather(v16, idx, dnums, (1,),
                          mode=lax.GatherScatterMode.PROMISE_IN_BOUNDS)

    def issue(i, s):
        lo = i * G
        r0, r1, r2, r3, _, gs, _ = bufs[s]
        pltpu.async_copy(table.at[i00v.at[pl.ds(lo, G)]], r0, gs)
        pltpu.async_copy(table.at[i01v.at[pl.ds(lo, G)]], r1, gs)
        pltpu.async_copy(table.at[i10v.at[pl.ds(lo, G)]], r2, gs)
        pltpu.async_copy(table.at[i11v.at[pl.ds(lo, G)]], r3, gs)

    def gdrain(s):
        r0, r1, r2, r3, _, gs, _ = bufs[s]
        for r in (r0, r1, r2, r3):
            pltpu.make_async_copy(table.at[pl.ds(0, G)], r, gs).wait()

    def wissue(i, s):
        _, _, _, _, ob, _, ws = bufs[s]
        pltpu.async_copy(ob, out.at[pl.ds(wbase + i * G, G)], ws)

    def wdrain(s):
        _, _, _, _, ob, _, ws = bufs[s]
        pltpu.make_async_copy(ob, out.at[pl.ds(0, G)], ws).wait()

    def compute(i, s):
        lo = i * G
        r0, r1, r2, r3, ob, _, _ = bufs[s]

        def group(q, _):
            gb = q * 16
            wa16 = w00v[pl.ds(lo + gb, 16)]
            wb16 = w01v[pl.ds(lo + gb, 16)]
            wc16 = w10v[pl.ds(lo + gb, 16)]
            wd16 = w11v[pl.ds(lo + gb, 16)]

            def row(g, _):
                rr = gb + g
                a = bcast(wa16, g)
                bq = bcast(wb16, g)
                cq = bcast(wc16, g)
                dq = bcast(wd16, g)
                for c in range(C // 16):
                    sl = pl.ds(c * 16, 16)
                    ob[rr, sl] = (a * r0[rr, sl] + bq * r1[rr, sl]
                                  + cq * r2[rr, sl] + dq * r3[rr, sl])
                return 0

            lax.fori_loop(0, 16, row, 0, unroll=False)
            return 0

        lax.fori_loop(0, G // 16, group, 0, unroll=False)

    issue(0, 0)

    @pl.loop(0, NCH, step=2)
    def pair(i):
        issue(i + 1, 1)
        gdrain(0)
        pl.when(i > 0)(lambda: wdrain(0))
        compute(i, 0)
        wissue(i, 0)
        pl.when(i + 2 < NCH)(lambda: issue(i + 2, 0))
        gdrain(1)
        pl.when(i > 0)(lambda: wdrain(1))
        compute(i + 1, 1)
        wissue(i + 1, 1)

    wdrain(0)
    wdrain(1)


@functools.lru_cache(maxsize=None)
def _get_sc_gather():
    return pl.kernel(
        _sc_gather_body,
        out_type=jax.ShapeDtypeStruct((ROWS, C), jnp.float32),
        mesh=plsc.VectorSubcoreMesh(core_axis_name="c", subcore_axis_name="s"),
        scratch_types=[
            pltpu.VMEM((RPW,), jnp.int32),
            pltpu.VMEM((RPW,), jnp.int32),
            pltpu.VMEM((RPW,), jnp.int32),
            pltpu.VMEM((RPW,), jnp.int32),
            pltpu.VMEM((RPW,), jnp.float32),
            pltpu.VMEM((RPW,), jnp.float32),
            pltpu.VMEM((RPW,), jnp.float32),
            pltpu.VMEM((RPW,), jnp.float32),
            pltpu.VMEM((G, C), jnp.float32),
            pltpu.VMEM((G, C), jnp.float32),
            pltpu.VMEM((G, C), jnp.float32),
            pltpu.VMEM((G, C), jnp.float32),
            pltpu.VMEM((G, C), jnp.float32),
            pltpu.VMEM((G, C), jnp.float32),
            pltpu.VMEM((G, C), jnp.float32),
            pltpu.VMEM((G, C), jnp.float32),
            pltpu.VMEM((G, C), jnp.float32),
            pltpu.VMEM((G, C), jnp.float32),
            pltpu.SemaphoreType.DMA,
            pltpu.SemaphoreType.DMA,
            pltpu.SemaphoreType.DMA,
            pltpu.SemaphoreType.DMA,
        ],
    )


def kernel(metadata, boxes, feat_p2, feat_p3, feat_p4, feat_p5):
    boxes2d = boxes[0]
    idx_w = _compute_indices(metadata, boxes2d)
    flats = [f.reshape(-1, C) for f in (feat_p2, feat_p3, feat_p4, feat_p5)]
    table = jnp.concatenate(flats, axis=0)
    args = [a[:, :NS].reshape(ROWS) for a in idx_w]
    pooled = _get_sc_gather()(table, *args)
    return pooled.reshape(1, N, GPB, SPG, C)[:, :, :, :PW].reshape(1, N, PH, PW, C)
